# baseline (speedup 1.0000x reference)
import functools

import jax
import jax.numpy as jnp
from jax import lax
from jax.experimental import pallas as pl
from jax.experimental.pallas import tpu as pltpu

N = 100000
C = 256
G = 64
H = 16

BLK1 = 20000
NBLK1 = N // BLK1
BLK2 = 10000
NBLK2 = N // BLK2


def _pool_mlp_kernel(x_ref, b_ref, w1_ref, w2_ref, scale_ref, acc_ref, cnt_ref):
    i = pl.program_id(0)

    @pl.when(i == 0)
    def _init():
        acc_ref[...] = jnp.zeros_like(acc_ref)
        cnt_ref[...] = jnp.zeros_like(cnt_ref)

    seg = b_ref[0, 0, :]
    gids = lax.broadcasted_iota(jnp.int32, (G, BLK1), 0)
    onehot = (gids == seg[None, :]).astype(jnp.float32)
    acc_ref[...] += jax.lax.dot_general(
        onehot, x_ref[...], (((1,), (0,)), ((), ())),
        preferred_element_type=jnp.float32)
    cnt_ref[...] += jnp.sum(onehot, axis=1, keepdims=True)

    @pl.when(i == NBLK1 - 1)
    def _finish():
        counts = jnp.maximum(cnt_ref[...], 1.0)
        mean = acc_ref[...] / counts
        h = jax.lax.dot_general(mean, w1_ref[...], (((1,), (1,)), ((), ())),
                                preferred_element_type=jnp.float32)
        h = jnp.maximum(h, 0.0)
        logits = jax.lax.dot_general(h, w2_ref[...], (((1,), (1,)), ((), ())),
                                     preferred_element_type=jnp.float32)
        scale_ref[...] = jax.nn.sigmoid(logits)


def _scale_kernel(x_ref, b_ref, scale_ref, out_ref):
    seg = b_ref[0, 0, :]
    gids = lax.broadcasted_iota(jnp.int32, (BLK2, G), 1)
    onehot = (gids == seg[:, None]).astype(jnp.float32)
    rows = jax.lax.dot_general(onehot, scale_ref[...], (((1,), (0,)), ((), ())),
                               preferred_element_type=jnp.float32)
    out_ref[...] = x_ref[...] * rows


def kernel(x, batch, W1, W2):
    b32 = batch.astype(jnp.int32)
    b1 = b32.reshape(NBLK1, 1, BLK1)
    b2 = b32.reshape(NBLK2, 1, BLK2)

    scale = pl.pallas_call(
        _pool_mlp_kernel,
        grid=(NBLK1,),
        in_specs=[
            pl.BlockSpec((BLK1, C), lambda i: (i, 0)),
            pl.BlockSpec((1, 1, BLK1), lambda i: (i, 0, 0)),
            pl.BlockSpec((H, C), lambda i: (0, 0)),
            pl.BlockSpec((C, H), lambda i: (0, 0)),
        ],
        out_specs=pl.BlockSpec((G, C), lambda i: (0, 0)),
        out_shape=jax.ShapeDtypeStruct((G, C), jnp.float32),
        scratch_shapes=[
            pltpu.VMEM((G, C), jnp.float32),
            pltpu.VMEM((G, 1), jnp.float32),
        ],
    )(x, b1, W1, W2)

    out = pl.pallas_call(
        _scale_kernel,
        grid=(NBLK2,),
        in_specs=[
            pl.BlockSpec((BLK2, C), lambda i: (i, 0)),
            pl.BlockSpec((1, 1, BLK2), lambda i: (i, 0, 0)),
            pl.BlockSpec((G, C), lambda i: (0, 0)),
        ],
        out_specs=pl.BlockSpec((BLK2, C), lambda i: (i, 0)),
        out_shape=jax.ShapeDtypeStruct((N, C), jnp.float32),
    )(x, b2, scale)
    return out
